# Initial kernel scaffold; baseline (speedup 1.0000x reference)
import jax
import jax.numpy as jnp
from jax.experimental import pallas as pl


def kernel(logits, logits2, labels, epoch, index):
    raise NotImplementedError("write your pallas kernel here")


# speedup vs baseline: 5.9472x; 5.9472x over previous
import jax
import jax.numpy as jnp
import numpy as np
from jax.experimental import pallas as pl
from jax.experimental.pallas import tpu as pltpu

_BATCH = 16384
_CLS = 1000
_NUM_CLEAN = 64
_FORGET = 0.2
_GRADUAL = 10
_EPOCHS = 100


def _sched():
    rs = np.ones(_EPOCHS) * _FORGET
    rs[:_GRADUAL] = np.linspace(0.0, _FORGET, _GRADUAL)
    return rs


_K = int((1.0 - _sched()[5]) * _BATCH)

_R = 256


def _ce_body(x1_ref, x2_ref, lab_ref, l1_ref, l2_ref):
    lab = lab_ref[...]
    col = jax.lax.broadcasted_iota(jnp.int32, (_R, _CLS), 1)
    onehot = col == lab
    for x_ref, out_ref in ((x1_ref, l1_ref), (x2_ref, l2_ref)):
        x = x_ref[...]
        m = jnp.max(x, axis=1, keepdims=True)
        s = jnp.sum(jnp.exp(x - m), axis=1, keepdims=True)
        xl = jnp.sum(jnp.where(onehot, x, 0.0), axis=1, keepdims=True)
        out_ref[...] = (m + jnp.log(s)) - xl


def _ce_losses(logits, logits2, labels2d):
    grid = _BATCH // _R
    return pl.pallas_call(
        _ce_body,
        grid=(grid,),
        in_specs=[
            pl.BlockSpec((_R, _CLS), lambda i: (i, 0)),
            pl.BlockSpec((_R, _CLS), lambda i: (i, 0)),
            pl.BlockSpec((_R, 1), lambda i: (i, 0)),
        ],
        out_specs=[
            pl.BlockSpec((_R, 1), lambda i: (i, 0)),
            pl.BlockSpec((_R, 1), lambda i: (i, 0)),
        ],
        out_shape=[
            jax.ShapeDtypeStruct((_BATCH, 1), jnp.float32),
            jax.ShapeDtypeStruct((_BATCH, 1), jnp.float32),
        ],
    )(logits, logits2, labels2d)


def _radix_select(bits, pos, k):
    shape = bits.shape
    sel0 = jnp.zeros(shape, dtype=jnp.int32)
    cand0 = jnp.ones(shape, dtype=jnp.int32)

    def step(src, nbits):
        def body(j, carry):
            sel, cand, r = carry
            b = nbits - 1 - j
            bit = jnp.bitwise_and(jax.lax.shift_right_logical(src, b), 1)
            zero = cand & (bit ^ 1)
            c = jnp.sum(zero)
            take_zero = r <= c
            sel = jnp.where(take_zero, sel, sel | zero)
            cand = jnp.where(take_zero, zero, cand & bit)
            r = jnp.where(take_zero, r, r - c)
            return sel, cand, r

        return body

    carry = (sel0, cand0, jnp.int32(k))
    carry = jax.lax.fori_loop(0, 32, step(bits, 32), carry)
    carry = jax.lax.fori_loop(0, 14, step(pos, 14), carry)
    sel, cand, _ = carry
    return (sel | cand) == 1


def _sel_body(l1_ref, l2_ref, idx_ref, s1_ref, s2_ref):
    l1 = l1_ref[...]
    l2 = l2_ref[...]
    filt = idx_ref[...] < _NUM_CLEAN
    row = jax.lax.broadcasted_iota(jnp.int32, l1.shape, 0)
    col = jax.lax.broadcasted_iota(jnp.int32, l1.shape, 1)
    pos = row * l1.shape[1] + col
    sel1 = _radix_select(jax.lax.bitcast_convert_type(l1, jnp.int32), pos, _K)
    sel2 = _radix_select(jax.lax.bitcast_convert_type(l2, jnp.int32), pos, _K)
    s1_ref[...] = jnp.sum(jnp.where(sel2 & filt, l1, 0.0))[None, None]
    s2_ref[...] = jnp.sum(jnp.where(sel1 & filt, l2, 0.0))[None, None]


def _select_sums(loss1, loss2, idx):
    return pl.pallas_call(
        _sel_body,
        out_shape=[
            jax.ShapeDtypeStruct((1, 1), jnp.float32),
            jax.ShapeDtypeStruct((1, 1), jnp.float32),
        ],
    )(loss1, loss2, idx)


def kernel(logits, logits2, labels, epoch, index):
    labels2d = labels.reshape(_BATCH, 1)
    loss1, loss2 = _ce_losses(logits, logits2, labels2d)
    s1, s2 = _select_sums(
        loss1.reshape(128, 128), loss2.reshape(128, 128), index.reshape(128, 128)
    )
    rs = jnp.asarray(_sched(), dtype=jnp.float32)
    num_remember_t = jnp.floor((1.0 - rs[epoch]) * _BATCH)
    return (s1[0, 0] / num_remember_t, s2[0, 0] / num_remember_t)


# speedup vs baseline: 6.4683x; 1.0876x over previous
import jax
import jax.numpy as jnp
import numpy as np
from jax.experimental import pallas as pl
from jax.experimental.pallas import tpu as pltpu

_BATCH = 16384
_CLS = 1000
_NUM_CLEAN = 64
_FORGET = 0.2
_GRADUAL = 10
_EPOCHS = 100


def _sched():
    rs = np.ones(_EPOCHS) * _FORGET
    rs[:_GRADUAL] = np.linspace(0.0, _FORGET, _GRADUAL)
    return rs


_K = int((1.0 - _sched()[5]) * _BATCH)

_R = 512


def _ce_body(x1_ref, x2_ref, lab_ref, l1_ref, l2_ref):
    lab = lab_ref[...]
    col = jax.lax.broadcasted_iota(jnp.int32, (_R, _CLS), 1)
    onehot = col == lab
    for x_ref, out_ref in ((x1_ref, l1_ref), (x2_ref, l2_ref)):
        x = x_ref[...]
        m = jnp.max(x, axis=1, keepdims=True)
        s = jnp.sum(jnp.exp(x - m), axis=1, keepdims=True)
        xl = jnp.sum(jnp.where(onehot, x, 0.0), axis=1, keepdims=True)
        out_ref[...] = (m + jnp.log(s)) - xl


def _ce_losses(logits, logits2, labels2d):
    grid = _BATCH // _R
    return pl.pallas_call(
        _ce_body,
        grid=(grid,),
        in_specs=[
            pl.BlockSpec((_R, _CLS), lambda i: (i, 0)),
            pl.BlockSpec((_R, _CLS), lambda i: (i, 0)),
            pl.BlockSpec((_R, 1), lambda i: (i, 0)),
        ],
        out_specs=[
            pl.BlockSpec((_R, 1), lambda i: (i, 0)),
            pl.BlockSpec((_R, 1), lambda i: (i, 0)),
        ],
        out_shape=[
            jax.ShapeDtypeStruct((_BATCH, 1), jnp.float32),
            jax.ShapeDtypeStruct((_BATCH, 1), jnp.float32),
        ],
    )(logits, logits2, labels2d)


def _radix_select(bits, pos, k):
    shape = bits.shape
    sel0 = jnp.zeros(shape, dtype=jnp.int32)
    cand0 = jnp.ones(shape, dtype=jnp.int32)

    def step(src, nbits):
        def body(j, carry):
            sel, cand, r = carry
            b = nbits - 1 - j
            bit = jnp.bitwise_and(jax.lax.shift_right_logical(src, b), 1)
            zero = cand & (bit ^ 1)
            c = jnp.sum(zero)
            take_zero = r <= c
            sel = jnp.where(take_zero, sel, sel | zero)
            cand = jnp.where(take_zero, zero, cand & bit)
            r = jnp.where(take_zero, r, r - c)
            return sel, cand, r

        return body

    carry = (sel0, cand0, jnp.int32(k))
    carry = jax.lax.fori_loop(0, 32, step(bits, 32), carry)
    carry = jax.lax.fori_loop(0, 14, step(pos, 14), carry)
    sel, cand, _ = carry
    return (sel | cand) == 1


def _sel_body(l1_ref, l2_ref, idx_ref, s1_ref, s2_ref):
    l1 = l1_ref[...]
    l2 = l2_ref[...]
    filt = idx_ref[...] < _NUM_CLEAN
    row = jax.lax.broadcasted_iota(jnp.int32, l1.shape, 0)
    col = jax.lax.broadcasted_iota(jnp.int32, l1.shape, 1)
    pos = row * l1.shape[1] + col
    sel1 = _radix_select(jax.lax.bitcast_convert_type(l1, jnp.int32), pos, _K)
    sel2 = _radix_select(jax.lax.bitcast_convert_type(l2, jnp.int32), pos, _K)
    s1_ref[...] = jnp.sum(jnp.where(sel2 & filt, l1, 0.0))[None, None]
    s2_ref[...] = jnp.sum(jnp.where(sel1 & filt, l2, 0.0))[None, None]


def _select_sums(loss1, loss2, idx):
    return pl.pallas_call(
        _sel_body,
        out_shape=[
            jax.ShapeDtypeStruct((1, 1), jnp.float32),
            jax.ShapeDtypeStruct((1, 1), jnp.float32),
        ],
    )(loss1, loss2, idx)


def kernel(logits, logits2, labels, epoch, index):
    labels2d = labels.reshape(_BATCH, 1)
    loss1, loss2 = _ce_losses(logits, logits2, labels2d)
    s1, s2 = _select_sums(
        loss1.reshape(128, 128), loss2.reshape(128, 128), index.reshape(128, 128)
    )
    rs = jnp.asarray(_sched(), dtype=jnp.float32)
    num_remember_t = jnp.floor((1.0 - rs[epoch]) * _BATCH)
    return (s1[0, 0] / num_remember_t, s2[0, 0] / num_remember_t)


# speedup vs baseline: 6.7128x; 1.0378x over previous
import jax
import jax.numpy as jnp
import numpy as np
from jax.experimental import pallas as pl
from jax.experimental.pallas import tpu as pltpu

_BATCH = 16384
_CLS = 1000
_NUM_CLEAN = 64
_FORGET = 0.2
_GRADUAL = 10
_EPOCHS = 100


def _sched():
    rs = np.ones(_EPOCHS) * _FORGET
    rs[:_GRADUAL] = np.linspace(0.0, _FORGET, _GRADUAL)
    return rs


_K = int((1.0 - _sched()[5]) * _BATCH)

_R = 1024


def _ce_body(x1_ref, x2_ref, lab_ref, l1_ref, l2_ref):
    lab = lab_ref[...]
    col = jax.lax.broadcasted_iota(jnp.int32, (_R, _CLS), 1)
    onehot = col == lab
    for x_ref, out_ref in ((x1_ref, l1_ref), (x2_ref, l2_ref)):
        x = x_ref[...]
        m = jnp.max(x, axis=1, keepdims=True)
        s = jnp.sum(jnp.exp(x - m), axis=1, keepdims=True)
        xl = jnp.sum(jnp.where(onehot, x, 0.0), axis=1, keepdims=True)
        out_ref[...] = (m + jnp.log(s)) - xl


def _ce_losses(logits, logits2, labels2d):
    grid = _BATCH // _R
    return pl.pallas_call(
        _ce_body,
        grid=(grid,),
        in_specs=[
            pl.BlockSpec((_R, _CLS), lambda i: (i, 0)),
            pl.BlockSpec((_R, _CLS), lambda i: (i, 0)),
            pl.BlockSpec((_R, 1), lambda i: (i, 0)),
        ],
        out_specs=[
            pl.BlockSpec((_R, 1), lambda i: (i, 0)),
            pl.BlockSpec((_R, 1), lambda i: (i, 0)),
        ],
        out_shape=[
            jax.ShapeDtypeStruct((_BATCH, 1), jnp.float32),
            jax.ShapeDtypeStruct((_BATCH, 1), jnp.float32),
        ],
    )(logits, logits2, labels2d)


def _radix_select(bits, pos, k):
    shape = bits.shape
    sel0 = jnp.zeros(shape, dtype=jnp.int32)
    cand0 = jnp.ones(shape, dtype=jnp.int32)

    def step(src, nbits):
        def body(j, carry):
            sel, cand, r = carry
            b = nbits - 1 - j
            bit = jnp.bitwise_and(jax.lax.shift_right_logical(src, b), 1)
            zero = cand & (bit ^ 1)
            c = jnp.sum(zero)
            take_zero = r <= c
            sel = jnp.where(take_zero, sel, sel | zero)
            cand = jnp.where(take_zero, zero, cand & bit)
            r = jnp.where(take_zero, r, r - c)
            return sel, cand, r

        return body

    carry = (sel0, cand0, jnp.int32(k))
    carry = jax.lax.fori_loop(0, 32, step(bits, 32), carry)
    carry = jax.lax.fori_loop(0, 14, step(pos, 14), carry)
    sel, cand, _ = carry
    return (sel | cand) == 1


def _sel_body(l1_ref, l2_ref, idx_ref, s1_ref, s2_ref):
    l1 = l1_ref[...]
    l2 = l2_ref[...]
    filt = idx_ref[...] < _NUM_CLEAN
    row = jax.lax.broadcasted_iota(jnp.int32, l1.shape, 0)
    col = jax.lax.broadcasted_iota(jnp.int32, l1.shape, 1)
    pos = row * l1.shape[1] + col
    sel1 = _radix_select(jax.lax.bitcast_convert_type(l1, jnp.int32), pos, _K)
    sel2 = _radix_select(jax.lax.bitcast_convert_type(l2, jnp.int32), pos, _K)
    s1_ref[...] = jnp.sum(jnp.where(sel2 & filt, l1, 0.0))[None, None]
    s2_ref[...] = jnp.sum(jnp.where(sel1 & filt, l2, 0.0))[None, None]


def _select_sums(loss1, loss2, idx):
    return pl.pallas_call(
        _sel_body,
        out_shape=[
            jax.ShapeDtypeStruct((1, 1), jnp.float32),
            jax.ShapeDtypeStruct((1, 1), jnp.float32),
        ],
    )(loss1, loss2, idx)


def kernel(logits, logits2, labels, epoch, index):
    labels2d = labels.reshape(_BATCH, 1)
    loss1, loss2 = _ce_losses(logits, logits2, labels2d)
    s1, s2 = _select_sums(
        loss1.reshape(128, 128), loss2.reshape(128, 128), index.reshape(128, 128)
    )
    rs = jnp.asarray(_sched(), dtype=jnp.float32)
    num_remember_t = jnp.floor((1.0 - rs[epoch]) * _BATCH)
    return (s1[0, 0] / num_remember_t, s2[0, 0] / num_remember_t)
